# baseline (speedup 1.0000x reference)
import jax
import jax.numpy as jnp
from jax.experimental import pallas as pl

N_EXPERTS = 64
K = 8
HID = 2048
BLK = 512


def _router_kernel(x_ref, w_ref, logits_ref, scores_ref, idx_ref):
    x = x_ref[...]
    w = w_ref[...]
    logits = jax.lax.dot_general(
        x, w, (((1,), (1,)), ((), ())), preferred_element_type=jnp.float32
    )
    logits_ref[...] = logits
    m = jnp.max(logits, axis=1, keepdims=True)
    e = jnp.exp(logits - m)
    p = e / jnp.sum(e, axis=1, keepdims=True)

    iota = jax.lax.broadcasted_iota(jnp.int32, p.shape, 1)
    vals = []
    idxs = []
    cur = p
    for _ in range(K):
        mv = jnp.max(cur, axis=1, keepdims=True)
        mi = jnp.min(jnp.where(cur == mv, iota, N_EXPERTS), axis=1, keepdims=True)
        vals.append(mv)
        idxs.append(mi)
        cur = jnp.where(iota == mi, -1.0, cur)
    v = jnp.concatenate(vals, axis=1)
    scores_ref[...] = v / jnp.sum(v, axis=1, keepdims=True)
    idx_ref[...] = jnp.concatenate(idxs, axis=1)


def kernel(hidden_states, weight):
    n = hidden_states.shape[0]
    outs = pl.pallas_call(
        _router_kernel,
        grid=(n // BLK,),
        in_specs=[
            pl.BlockSpec((BLK, HID), lambda i: (i, 0)),
            pl.BlockSpec((N_EXPERTS, HID), lambda i: (0, 0)),
        ],
        out_specs=[
            pl.BlockSpec((BLK, N_EXPERTS), lambda i: (i, 0)),
            pl.BlockSpec((BLK, K), lambda i: (i, 0)),
            pl.BlockSpec((BLK, K), lambda i: (i, 0)),
        ],
        out_shape=[
            jax.ShapeDtypeStruct((n, N_EXPERTS), jnp.float32),
            jax.ShapeDtypeStruct((n, K), jnp.float32),
            jax.ShapeDtypeStruct((n, K), jnp.int32),
        ],
    )(hidden_states, weight)
    return (outs[0], outs[1], outs[2])
